# baseline (speedup 1.0000x reference)
import functools

import jax
import jax.numpy as jnp
from jax import lax
from jax.experimental import pallas as pl
from jax.experimental.pallas import tpu as pltpu
from jax.experimental.pallas import tpu_sc as plsc

B = 16384
D = 64
NC = 2
NS = 16
NW = NC * NS
BPW = B // NW
CHUNK = 128
NCHUNK = BPW // CHUNK
NV = 1000000


DP = 2 * D
NPAIR = NV // 2
NBUF = 4

TW = 8064
TH = TW // 2
TQ = TW // 4
VCUT = NV - NV % 128
NTB = VCUT // TW
NPOUT = NTB * TQ
NTAIL = NV - VCUT


def _tp_eye():
    r = lax.broadcasted_iota(jnp.int32, (D, D), 0)
    c = lax.broadcasted_iota(jnp.int32, (D, D), 1)
    return (r == c).astype(jnp.float32)


def _tp_body(u_ref, i_ref, ou_ref, oi_ref):
    eye = _tp_eye()

    def transpose_pack(a):
        halves = []
        for q in range(2):
            t = lax.dot_general(a[:, q * TH:(q + 1) * TH], eye,
                                (((0,), (0,)), ((), ())),
                                preferred_element_type=jnp.float32)
            halves.append(pltpu.bitcast(t.astype(jnp.bfloat16), jnp.float32))
        return jnp.concatenate(halves, axis=1)

    ou_ref[...] = transpose_pack(u_ref[...])
    oi_ref[...] = transpose_pack(i_ref[...])


_tp = pl.pallas_call(
    _tp_body,
    grid=(NTB,),
    in_specs=[
        pl.BlockSpec((D, TW), lambda i: (0, i)),
        pl.BlockSpec((D, TW), lambda i: (0, i)),
    ],
    out_specs=[
        pl.BlockSpec((TQ, DP), lambda i: (i, 0)),
        pl.BlockSpec((TQ, DP), lambda i: (i, 0)),
    ],
    out_shape=[
        jax.ShapeDtypeStruct((NPOUT, DP), jnp.float32),
        jax.ShapeDtypeStruct((NPOUT, DP), jnp.float32),
    ],
    compiler_params=pltpu.CompilerParams(vmem_limit_bytes=60 * 1024 * 1024),
)


def _sc_gather2_body(uidx_hbm, iidx_hbm, utab_hbm, itab_hbm, uout_hbm, iout_hbm,
                     uidx_v, iidx_v, *rest):
    bufs = rest[:NBUF]
    sem_g = rest[NBUF:2 * NBUF]
    sem_o = rest[2 * NBUF:3 * NBUF]
    wid = lax.axis_index("s") * NC + lax.axis_index("c")
    base = wid * BPW
    pltpu.sync_copy(uidx_hbm.at[wid], uidx_v)
    pltpu.sync_copy(iidx_hbm.at[wid], iidx_v)

    jobs = [(uidx_v, utab_hbm, uout_hbm, c) for c in range(NCHUNK)]
    jobs += [(iidx_v, itab_hbm, iout_hbm, c) for c in range(NCHUNK)]
    njobs = len(jobs)

    def issue(j):
        idx_v, tab, _, c = jobs[j]
        return pltpu.async_copy(tab.at[idx_v.at[c]], bufs[j % NBUF], sem_g[j % NBUF])

    gh = [issue(j) for j in range(NBUF)]
    oh = [None] * njobs
    for j in range(njobs):
        gh[j].wait()
        _, _, out, c = jobs[j]
        oh[j] = pltpu.async_copy(
            bufs[j % NBUF], out.at[pl.ds(base + c * CHUNK, CHUNK)], sem_o[j % NBUF])
        if j + NBUF < njobs:
            oh[j].wait()
            gh.append(issue(j + NBUF))
    for j in range(njobs - NBUF, njobs):
        oh[j].wait()


@functools.cache
def _sc_gather2():
    mesh = plsc.VectorSubcoreMesh(
        core_axis_name="c", subcore_axis_name="s", num_cores=NC, num_subcores=NS
    )
    scratch = [
        pltpu.VMEM((NCHUNK, CHUNK), jnp.int32),
        pltpu.VMEM((NCHUNK, CHUNK), jnp.int32),
    ]
    scratch += [pltpu.VMEM((CHUNK, DP), jnp.float32) for _ in range(NBUF)]
    scratch += [pltpu.SemaphoreType.DMA for _ in range(2 * NBUF)]
    return pl.kernel(
        _sc_gather2_body,
        out_type=[
            jax.ShapeDtypeStruct((B, DP), jnp.float32),
            jax.ShapeDtypeStruct((B, DP), jnp.float32),
        ],
        mesh=mesh,
        scratch_types=scratch,
    )


BT = 2048


def _mlp_body(xu_ref, xi_ref, pu_ref, pi_ref, ohu_ref, ohi_ref,
              tailu_ref, taili_ref, w1u_ref, w1i_ref, b1_ref,
              w2_ref, b2_ref, w3_ref, b3_ref, out_ref):
    xu = xu_ref[...]
    xi = xi_ref[...]
    def select_quarter(x, p):
        v = jnp.where(p >= 2, x[:, D:], x[:, :D])
        vi = lax.bitcast_convert_type(v, jnp.int32)
        lo = lax.bitcast_convert_type(lax.shift_left(vi, 16), jnp.float32)
        hi = lax.bitcast_convert_type(
            lax.bitwise_and(vi, jnp.int32(-65536)), jnp.float32)
        return jnp.where(p % 2 == 0, lo, hi)

    u = select_quarter(xu, pu_ref[...])
    i = select_quarter(xi, pi_ref[...])
    u = jnp.where(pu_ref[...] < 0,
                  jnp.dot(ohu_ref[...], tailu_ref[...],
                          preferred_element_type=jnp.float32), u)
    i = jnp.where(pi_ref[...] < 0,
                  jnp.dot(ohi_ref[...], taili_ref[...],
                          preferred_element_type=jnp.float32), i)
    h = jnp.dot(u, w1u_ref[...], preferred_element_type=jnp.float32)
    h += jnp.dot(i, w1i_ref[...], preferred_element_type=jnp.float32)
    h = jnp.maximum(h + b1_ref[...], 0.0)
    h2 = jnp.dot(h, w2_ref[...], preferred_element_type=jnp.float32)
    h2 = jnp.maximum(h2 + b2_ref[...], 0.0)
    z = jnp.sum(h2 * w3_ref[...], axis=-1) + b3_ref[0, 0]
    out_ref[...] = 1.0 / (1.0 + jnp.exp(-z))


_mlp = pl.pallas_call(
    _mlp_body,
    grid=(B // BT,),
    in_specs=[
        pl.BlockSpec((BT, DP), lambda i: (i, 0)),
        pl.BlockSpec((BT, DP), lambda i: (i, 0)),
        pl.BlockSpec((BT, 1), lambda i: (i, 0)),
        pl.BlockSpec((BT, 1), lambda i: (i, 0)),
        pl.BlockSpec((BT, D), lambda i: (i, 0)),
        pl.BlockSpec((BT, D), lambda i: (i, 0)),
        pl.BlockSpec((NTAIL, D), lambda i: (0, 0)),
        pl.BlockSpec((NTAIL, D), lambda i: (0, 0)),
        pl.BlockSpec((D, 32), lambda i: (0, 0)),
        pl.BlockSpec((D, 32), lambda i: (0, 0)),
        pl.BlockSpec((1, 32), lambda i: (0, 0)),
        pl.BlockSpec((32, 16), lambda i: (0, 0)),
        pl.BlockSpec((1, 16), lambda i: (0, 0)),
        pl.BlockSpec((1, 16), lambda i: (0, 0)),
        pl.BlockSpec((1, 1), lambda i: (0, 0)),
    ],
    out_specs=pl.BlockSpec((BT,), lambda i: (i,)),
    out_shape=jax.ShapeDtypeStruct((B,), jnp.float32),
)


def kernel(user_indices, item_indices, emb_user, emb_item, W1, b1, W2, b2, W3, b3):
    ui = user_indices.astype(jnp.int32)
    ii = item_indices.astype(jnp.int32)

    def pair_par(idx):
        blk = idx // TW
        w = idx % TW
        half = w // TH
        pr = (w % TH) // 2
        slot = w % 2
        par = half * 2 + slot
        pair = blk * TQ + pr
        pair = jnp.where(idx >= VCUT, 0, pair)
        par = jnp.where(idx >= VCUT, -1, par)
        return pair, par

    upair, pu = pair_par(ui)
    ipair, pi = pair_par(ii)
    upair = upair.reshape(NW, NCHUNK, CHUNK)
    ipair = ipair.reshape(NW, NCHUNK, CHUNK)
    pu = pu.reshape(B, 1)
    pi = pi.reshape(B, 1)
    ut2, it2 = _tp(emb_user.T, emb_item.T)
    xu, xi = _sc_gather2()(upair, ipair, ut2, it2)
    ohu = jax.nn.one_hot(ui - VCUT, NTAIL, dtype=jnp.float32)
    ohi = jax.nn.one_hot(ii - VCUT, NTAIL, dtype=jnp.float32)
    w1u = W1[:, :D].T
    w1i = W1[:, D:].T
    return _mlp(xu, xi, pu, pi, ohu, ohi, emb_user[VCUT:], emb_item[VCUT:],
                w1u, w1i, b1.reshape(1, -1), W2.T,
                b2.reshape(1, -1), W3, b3.reshape(1, 1))


# speedup vs baseline: 1.1922x; 1.1922x over previous
import functools

import jax
import jax.numpy as jnp
from jax import lax
from jax.experimental import pallas as pl
from jax.experimental.pallas import tpu as pltpu
from jax.experimental.pallas import tpu_sc as plsc

B = 16384
D = 64
NC = 2
NS = 16
NW = NC * NS
BPW = B // NW
CHUNK = 128
NCHUNK = BPW // CHUNK
NV = 1000000


DP = 2 * D
NPAIR = NV // 2
NBUF = 4

TW = 16128
TH = TW // 2
TQ = TW // 4
VCUT = NV - NV % 128
NTB = VCUT // TW
NPOUT = NTB * TQ
NTAIL = NV - VCUT


def _tp_eye():
    r = lax.broadcasted_iota(jnp.int32, (D, D), 0)
    c = lax.broadcasted_iota(jnp.int32, (D, D), 1)
    return (r == c).astype(jnp.float32)


def _tp_body(u_ref, i_ref, ou_ref, oi_ref):
    eye = _tp_eye()

    def transpose_pack(a):
        halves = []
        for q in range(2):
            t = lax.dot_general(a[:, q * TH:(q + 1) * TH], eye,
                                (((0,), (0,)), ((), ())),
                                preferred_element_type=jnp.float32)
            halves.append(pltpu.bitcast(t.astype(jnp.bfloat16), jnp.float32))
        return jnp.concatenate(halves, axis=1)

    ou_ref[...] = transpose_pack(u_ref[...])
    oi_ref[...] = transpose_pack(i_ref[...])


_tp = pl.pallas_call(
    _tp_body,
    grid=(NTB,),
    in_specs=[
        pl.BlockSpec((D, TW), lambda i: (0, i)),
        pl.BlockSpec((D, TW), lambda i: (0, i)),
    ],
    out_specs=[
        pl.BlockSpec((TQ, DP), lambda i: (i, 0)),
        pl.BlockSpec((TQ, DP), lambda i: (i, 0)),
    ],
    out_shape=[
        jax.ShapeDtypeStruct((NPOUT, DP), jnp.float32),
        jax.ShapeDtypeStruct((NPOUT, DP), jnp.float32),
    ],
    compiler_params=pltpu.CompilerParams(vmem_limit_bytes=60 * 1024 * 1024),
)


def _sc_gather2_body(uidx_hbm, iidx_hbm, utab_hbm, itab_hbm, uout_hbm, iout_hbm,
                     uidx_v, iidx_v, *rest):
    bufs = rest[:NBUF]
    sem_g = rest[NBUF:2 * NBUF]
    sem_o = rest[2 * NBUF:3 * NBUF]
    wid = lax.axis_index("s") * NC + lax.axis_index("c")
    base = wid * BPW
    pltpu.sync_copy(uidx_hbm.at[wid], uidx_v)
    pltpu.sync_copy(iidx_hbm.at[wid], iidx_v)

    jobs = [(uidx_v, utab_hbm, uout_hbm, c) for c in range(NCHUNK)]
    jobs += [(iidx_v, itab_hbm, iout_hbm, c) for c in range(NCHUNK)]
    njobs = len(jobs)

    def issue(j):
        idx_v, tab, _, c = jobs[j]
        return pltpu.async_copy(tab.at[idx_v.at[c]], bufs[j % NBUF], sem_g[j % NBUF])

    gh = [issue(j) for j in range(NBUF)]
    oh = [None] * njobs
    for j in range(njobs):
        gh[j].wait()
        _, _, out, c = jobs[j]
        oh[j] = pltpu.async_copy(
            bufs[j % NBUF], out.at[pl.ds(base + c * CHUNK, CHUNK)], sem_o[j % NBUF])
        if j + NBUF < njobs:
            oh[j].wait()
            gh.append(issue(j + NBUF))
    for j in range(njobs - NBUF, njobs):
        oh[j].wait()


@functools.cache
def _sc_gather2():
    mesh = plsc.VectorSubcoreMesh(
        core_axis_name="c", subcore_axis_name="s", num_cores=NC, num_subcores=NS
    )
    scratch = [
        pltpu.VMEM((NCHUNK, CHUNK), jnp.int32),
        pltpu.VMEM((NCHUNK, CHUNK), jnp.int32),
    ]
    scratch += [pltpu.VMEM((CHUNK, DP), jnp.float32) for _ in range(NBUF)]
    scratch += [pltpu.SemaphoreType.DMA for _ in range(2 * NBUF)]
    return pl.kernel(
        _sc_gather2_body,
        out_type=[
            jax.ShapeDtypeStruct((B, DP), jnp.float32),
            jax.ShapeDtypeStruct((B, DP), jnp.float32),
        ],
        mesh=mesh,
        scratch_types=scratch,
    )


BT = 2048


def _mlp_body(xu_ref, xi_ref, pu_ref, pi_ref, ohu_ref, ohi_ref,
              tailu_ref, taili_ref, w1u_ref, w1i_ref, b1_ref,
              w2_ref, b2_ref, w3_ref, b3_ref, out_ref):
    xu = xu_ref[...]
    xi = xi_ref[...]
    def select_quarter(x, p):
        v = jnp.where(p >= 2, x[:, D:], x[:, :D])
        vi = lax.bitcast_convert_type(v, jnp.int32)
        lo = lax.bitcast_convert_type(lax.shift_left(vi, 16), jnp.float32)
        hi = lax.bitcast_convert_type(
            lax.bitwise_and(vi, jnp.int32(-65536)), jnp.float32)
        return jnp.where(p % 2 == 0, lo, hi)

    u = select_quarter(xu, pu_ref[...])
    i = select_quarter(xi, pi_ref[...])
    u = jnp.where(pu_ref[...] < 0,
                  jnp.dot(ohu_ref[...], tailu_ref[...],
                          preferred_element_type=jnp.float32), u)
    i = jnp.where(pi_ref[...] < 0,
                  jnp.dot(ohi_ref[...], taili_ref[...],
                          preferred_element_type=jnp.float32), i)
    h = jnp.dot(u, w1u_ref[...], preferred_element_type=jnp.float32)
    h += jnp.dot(i, w1i_ref[...], preferred_element_type=jnp.float32)
    h = jnp.maximum(h + b1_ref[...], 0.0)
    h2 = jnp.dot(h, w2_ref[...], preferred_element_type=jnp.float32)
    h2 = jnp.maximum(h2 + b2_ref[...], 0.0)
    z = jnp.sum(h2 * w3_ref[...], axis=-1) + b3_ref[0, 0]
    out_ref[...] = 1.0 / (1.0 + jnp.exp(-z))


_mlp = pl.pallas_call(
    _mlp_body,
    grid=(B // BT,),
    in_specs=[
        pl.BlockSpec((BT, DP), lambda i: (i, 0)),
        pl.BlockSpec((BT, DP), lambda i: (i, 0)),
        pl.BlockSpec((BT, 1), lambda i: (i, 0)),
        pl.BlockSpec((BT, 1), lambda i: (i, 0)),
        pl.BlockSpec((BT, D), lambda i: (i, 0)),
        pl.BlockSpec((BT, D), lambda i: (i, 0)),
        pl.BlockSpec((NTAIL, D), lambda i: (0, 0)),
        pl.BlockSpec((NTAIL, D), lambda i: (0, 0)),
        pl.BlockSpec((D, 32), lambda i: (0, 0)),
        pl.BlockSpec((D, 32), lambda i: (0, 0)),
        pl.BlockSpec((1, 32), lambda i: (0, 0)),
        pl.BlockSpec((32, 16), lambda i: (0, 0)),
        pl.BlockSpec((1, 16), lambda i: (0, 0)),
        pl.BlockSpec((1, 16), lambda i: (0, 0)),
        pl.BlockSpec((1, 1), lambda i: (0, 0)),
    ],
    out_specs=pl.BlockSpec((BT,), lambda i: (i,)),
    out_shape=jax.ShapeDtypeStruct((B,), jnp.float32),
)


def kernel(user_indices, item_indices, emb_user, emb_item, W1, b1, W2, b2, W3, b3):
    ui = user_indices.astype(jnp.int32)
    ii = item_indices.astype(jnp.int32)

    def pair_par(idx):
        blk = idx // TW
        w = idx % TW
        half = w // TH
        pr = (w % TH) // 2
        slot = w % 2
        par = half * 2 + slot
        pair = blk * TQ + pr
        pair = jnp.where(idx >= VCUT, 0, pair)
        par = jnp.where(idx >= VCUT, -1, par)
        return pair, par

    upair, pu = pair_par(ui)
    ipair, pi = pair_par(ii)
    upair = upair.reshape(NW, NCHUNK, CHUNK)
    ipair = ipair.reshape(NW, NCHUNK, CHUNK)
    pu = pu.reshape(B, 1)
    pi = pi.reshape(B, 1)
    ut2, it2 = _tp(emb_user.T, emb_item.T)
    xu, xi = _sc_gather2()(upair, ipair, ut2, it2)
    ohu = jax.nn.one_hot(ui - VCUT, NTAIL, dtype=jnp.float32)
    ohi = jax.nn.one_hot(ii - VCUT, NTAIL, dtype=jnp.float32)
    w1u = W1[:, :D].T
    w1i = W1[:, D:].T
    return _mlp(xu, xi, pu, pi, ohu, ohi, emb_user[VCUT:], emb_item[VCUT:],
                w1u, w1i, b1.reshape(1, -1), W2.T,
                b2.reshape(1, -1), W3, b3.reshape(1, 1))
